# baseline (speedup 1.0000x reference)
import functools

import jax
import jax.numpy as jnp
from jax import lax
from jax.experimental import pallas as pl
from jax.experimental.pallas import tpu as pltpu
from jax.experimental.pallas import tpu_sc as plsc

E = 16
TOPK = 2
DIM = 2048
INTER = 1024
T = 4096
P = T * TOPK

BM = 128
NUM_BLOCKS = (P + E * (BM - 1) + BM - 1) // BM
PT = NUM_BLOCKS * BM

NC = 2
NS = 16
NW = NC * NS

RW = PT // NW
GC = 32
NGC = RW // GC

TW = T // NW
CT = 16
NCT = TW // CT


def _plan(indices, token_mask, weights):
    e_f = jnp.where(token_mask[:, None], indices, -1).reshape(P).astype(jnp.int32)
    valid = e_f >= 0
    e_c = jnp.clip(e_f, 0, E - 1)
    onehot = (e_f[:, None] == jnp.arange(E, dtype=jnp.int32)).astype(jnp.int32)
    cum = jnp.cumsum(onehot, axis=0)
    counts = cum[-1]
    rank = jnp.take_along_axis(cum, e_c[:, None], axis=1)[:, 0] - 1
    padded_counts = ((counts + BM - 1) // BM) * BM
    starts = jnp.concatenate(
        [jnp.zeros((1,), jnp.int32), jnp.cumsum(padded_counts)[:-1]])
    dest = starts[e_c] + rank
    dest_or_drop = jnp.where(valid, dest, PT)
    src_pair = jnp.full((PT,), -1, jnp.int32).at[dest_or_drop].set(
        jnp.arange(P, dtype=jnp.int32), mode="drop")
    src_row = jnp.where(src_pair >= 0, src_pair // TOPK, 0).astype(jnp.int32)
    pw = jnp.zeros((PT,), jnp.float32).at[dest_or_drop].set(
        weights.reshape(P), mode="drop")
    bid = jnp.arange(NUM_BLOCKS, dtype=jnp.int32) * BM
    block_expert = jnp.sum(
        (bid[:, None] >= starts[None, :]).astype(jnp.int32), axis=1) - 1
    pair_pos = jnp.where(valid, dest, PT - 1).reshape(T, TOPK)
    return src_row, block_expert, pair_pos, pw


@functools.lru_cache(maxsize=None)
def _build_dispatch():
    mesh = plsc.VectorSubcoreMesh(core_axis_name="c", subcore_axis_name="s")

    @functools.partial(
        pl.kernel,
        mesh=mesh,
        out_type=jax.ShapeDtypeStruct((PT, DIM), jnp.float32),
        scratch_types=[
            pltpu.VMEM((NGC, GC), jnp.int32),
            pltpu.VMEM((GC, DIM), jnp.float32),
            pltpu.SemaphoreType.DMA,
        ],
    )
    def _dispatch(x_hbm, idx_hbm, out_hbm, idx_v, buf_v, sem):
        wid = lax.axis_index("s") * NC + lax.axis_index("c")
        pltpu.sync_copy(idx_hbm.at[wid], idx_v)

        def chunk(c, carry):
            pltpu.async_copy(x_hbm.at[idx_v.at[c]], buf_v, sem).wait()
            pltpu.sync_copy(buf_v, out_hbm.at[pl.ds(wid * RW + c * GC, GC)])
            return carry

        lax.fori_loop(0, NGC, chunk, 0)

    return _dispatch


def _gemm_body(be_ref, a_ref, w1_ref, w2_ref, pw_ref, y_ref):
    a = a_ref[...].astype(jnp.bfloat16)
    h = jnp.dot(a, w1_ref[0], preferred_element_type=jnp.float32)
    gate = h[:, :INTER]
    up = h[:, INTER:]
    su = (gate * lax.logistic(gate)) * up * pw_ref[...]
    y_ref[...] = jnp.dot(su.astype(jnp.bfloat16), w2_ref[0],
                         preferred_element_type=jnp.float32)


def _grouped_gemm(block_expert, a, w1, w2, pw):
    grid_spec = pltpu.PrefetchScalarGridSpec(
        num_scalar_prefetch=1,
        grid=(NUM_BLOCKS,),
        in_specs=[
            pl.BlockSpec((BM, DIM), lambda i, be: (i, 0)),
            pl.BlockSpec((1, DIM, 2 * INTER), lambda i, be: (be[i], 0, 0)),
            pl.BlockSpec((1, INTER, DIM), lambda i, be: (be[i], 0, 0)),
            pl.BlockSpec((BM, 1), lambda i, be: (i, 0)),
        ],
        out_specs=pl.BlockSpec((BM, DIM), lambda i, be: (i, 0)),
    )
    return pl.pallas_call(
        _gemm_body,
        grid_spec=grid_spec,
        out_shape=jax.ShapeDtypeStruct((PT, DIM), jnp.float32),
        compiler_params=pltpu.CompilerParams(
            dimension_semantics=("arbitrary",)),
    )(block_expert, a, w1, w2, pw.reshape(PT, 1))


@functools.lru_cache(maxsize=None)
def _build_combine():
    mesh = plsc.VectorSubcoreMesh(core_axis_name="c", subcore_axis_name="s")

    @functools.partial(
        pl.kernel,
        mesh=mesh,
        out_type=jax.ShapeDtypeStruct((T, DIM), jnp.float32),
        scratch_types=[
            pltpu.VMEM((NCT, CT), jnp.int32),
            pltpu.VMEM((NCT, CT), jnp.int32),
            pltpu.VMEM((CT, DIM), jnp.float32),
            pltpu.VMEM((CT, DIM), jnp.float32),
            pltpu.SemaphoreType.DMA,
        ],
    )
    def _combine(y_hbm, p0_hbm, p1_hbm, out_hbm,
                 p0_v, p1_v, buf0, buf1, sem):
        wid = lax.axis_index("s") * NC + lax.axis_index("c")
        pltpu.sync_copy(p0_hbm.at[wid], p0_v)
        pltpu.sync_copy(p1_hbm.at[wid], p1_v)

        def chunk(c, carry):
            cp0 = pltpu.async_copy(y_hbm.at[p0_v.at[c]], buf0, sem)
            cp1 = pltpu.async_copy(y_hbm.at[p1_v.at[c]], buf1, sem)
            cp0.wait()
            cp1.wait()

            def tok(i, carry2):
                def vec(j, carry3):
                    a = buf0[i, pl.ds(j * 16, 16)]
                    b = buf1[i, pl.ds(j * 16, 16)]
                    buf0[i, pl.ds(j * 16, 16)] = a + b
                    return carry3

                lax.fori_loop(0, DIM // 16, vec, 0)
                return carry2

            lax.fori_loop(0, CT, tok, 0)
            pltpu.sync_copy(buf0, out_hbm.at[pl.ds(wid * TW + c * CT, CT)])
            return carry

        lax.fori_loop(0, NCT, chunk, 0)

    return _combine


def kernel(x, token_mask, weights, indices, gate_and_up_projs, down_projs):
    src_row, block_expert, pair_pos, pw = _plan(indices, token_mask, weights)
    a = _build_dispatch()(x, src_row.reshape(NW, NGC, GC))
    w1_16 = gate_and_up_projs.astype(jnp.bfloat16)
    w2_16 = down_projs.astype(jnp.bfloat16)
    y = _grouped_gemm(block_expert, a, w1_16, w2_16, pw)
    p0 = pair_pos[:, 0].reshape(NW, NCT, CT)
    p1 = pair_pos[:, 1].reshape(NW, NCT, CT)
    out = _build_combine()(y, p0, p1)
    return out


# speedup vs baseline: 1.3601x; 1.3601x over previous
import functools

import jax
import jax.numpy as jnp
from jax import lax
from jax.experimental import pallas as pl
from jax.experimental.pallas import tpu as pltpu
from jax.experimental.pallas import tpu_sc as plsc

E = 16
TOPK = 2
DIM = 2048
INTER = 1024
T = 4096
P = T * TOPK

BM = 128
NUM_BLOCKS = (P + E * (BM - 1) + BM - 1) // BM
PT = NUM_BLOCKS * BM

NC = 2
NS = 16
NW = NC * NS

RW = PT // NW
GC = 16
NGC = RW // GC

TW = T // NW
CT = 8
NCT = TW // CT


def _plan(indices, token_mask, weights):
    e_f = jnp.where(token_mask[:, None], indices, -1).reshape(P).astype(jnp.int32)
    valid = e_f >= 0
    e_c = jnp.clip(e_f, 0, E - 1)
    onehot = (e_f[:, None] == jnp.arange(E, dtype=jnp.int32)).astype(jnp.int32)
    cum = jnp.cumsum(onehot, axis=0)
    counts = cum[-1]
    rank = jnp.take_along_axis(cum, e_c[:, None], axis=1)[:, 0] - 1
    padded_counts = ((counts + BM - 1) // BM) * BM
    starts = jnp.concatenate(
        [jnp.zeros((1,), jnp.int32), jnp.cumsum(padded_counts)[:-1]])
    dest = starts[e_c] + rank
    dest_or_drop = jnp.where(valid, dest, PT)
    src_pair = jnp.full((PT,), -1, jnp.int32).at[dest_or_drop].set(
        jnp.arange(P, dtype=jnp.int32), mode="drop")
    src_row = jnp.where(src_pair >= 0, src_pair // TOPK, 0).astype(jnp.int32)
    pw = jnp.zeros((PT,), jnp.float32).at[dest_or_drop].set(
        weights.reshape(P), mode="drop")
    bid = jnp.arange(NUM_BLOCKS, dtype=jnp.int32) * BM
    block_expert = jnp.sum(
        (bid[:, None] >= starts[None, :]).astype(jnp.int32), axis=1) - 1
    pair_pos = jnp.where(valid, dest, PT - 1).reshape(T, TOPK)
    return src_row, block_expert, pair_pos, pw


@functools.lru_cache(maxsize=None)
def _build_dispatch():
    mesh = plsc.VectorSubcoreMesh(core_axis_name="c", subcore_axis_name="s")

    @functools.partial(
        pl.kernel,
        mesh=mesh,
        out_type=jax.ShapeDtypeStruct((PT, DIM), jnp.float32),
        scratch_types=[
            pltpu.VMEM((NGC, GC), jnp.int32),
            pltpu.VMEM((GC, DIM), jnp.float32),
            pltpu.VMEM((GC, DIM), jnp.float32),
            pltpu.SemaphoreType.DMA,
            pltpu.SemaphoreType.DMA,
            pltpu.SemaphoreType.DMA,
            pltpu.SemaphoreType.DMA,
        ],
    )
    def _dispatch(x_hbm, idx_hbm, out_hbm, idx_v, buf0, buf1,
                  g0, g1, w0, w1):
        wid = lax.axis_index("s") * NC + lax.axis_index("c")
        pltpu.sync_copy(idx_hbm.at[wid], idx_v)
        bufs, gsem, wsem = (buf0, buf1), (g0, g1), (w0, w1)
        gcp = [None, None]
        wcp = [None, None]
        gcp[0] = pltpu.async_copy(x_hbm.at[idx_v.at[0]], buf0, g0)
        for c in range(NGC):
            p, q = c & 1, (c + 1) & 1
            if c + 1 < NGC:
                if wcp[q] is not None:
                    wcp[q].wait()
                gcp[q] = pltpu.async_copy(
                    x_hbm.at[idx_v.at[c + 1]], bufs[q], gsem[q])
            gcp[p].wait()
            wcp[p] = pltpu.async_copy(
                bufs[p], out_hbm.at[pl.ds(wid * RW + c * GC, GC)], wsem[p])
        wcp[0].wait()
        wcp[1].wait()

    return _dispatch


def _gemm_body(be_ref, a_ref, w1_ref, w2_ref, pw_ref, y_ref):
    a = a_ref[...]
    h = jnp.dot(a, w1_ref[0], preferred_element_type=jnp.float32)
    gate = h[:, :INTER]
    up = h[:, INTER:]
    su = (gate * lax.logistic(gate)) * up * pw_ref[...]
    y_ref[...] = jnp.dot(su, w2_ref[0], preferred_element_type=jnp.float32)


def _grouped_gemm(block_expert, a, w1, w2, pw):
    grid_spec = pltpu.PrefetchScalarGridSpec(
        num_scalar_prefetch=1,
        grid=(NUM_BLOCKS,),
        in_specs=[
            pl.BlockSpec((BM, DIM), lambda i, be: (i, 0)),
            pl.BlockSpec((1, DIM, 2 * INTER), lambda i, be: (be[i], 0, 0)),
            pl.BlockSpec((1, INTER, DIM), lambda i, be: (be[i], 0, 0)),
            pl.BlockSpec((BM, 1), lambda i, be: (i, 0)),
        ],
        out_specs=pl.BlockSpec((BM, DIM), lambda i, be: (i, 0)),
    )
    return pl.pallas_call(
        _gemm_body,
        grid_spec=grid_spec,
        out_shape=jax.ShapeDtypeStruct((PT, DIM), jnp.float32),
        compiler_params=pltpu.CompilerParams(
            dimension_semantics=("arbitrary",)),
    )(block_expert, a, w1, w2, pw.reshape(PT, 1))


@functools.lru_cache(maxsize=None)
def _build_combine():
    mesh = plsc.VectorSubcoreMesh(core_axis_name="c", subcore_axis_name="s")

    @functools.partial(
        pl.kernel,
        mesh=mesh,
        out_type=jax.ShapeDtypeStruct((T, DIM), jnp.float32),
        scratch_types=[
            pltpu.VMEM((NCT, CT), jnp.int32),
            pltpu.VMEM((NCT, CT), jnp.int32),
            pltpu.VMEM((CT, DIM), jnp.float32),
            pltpu.VMEM((CT, DIM), jnp.float32),
            pltpu.VMEM((CT, DIM), jnp.float32),
            pltpu.VMEM((CT, DIM), jnp.float32),
            pltpu.SemaphoreType.DMA,
            pltpu.SemaphoreType.DMA,
            pltpu.SemaphoreType.DMA,
            pltpu.SemaphoreType.DMA,
            pltpu.SemaphoreType.DMA,
            pltpu.SemaphoreType.DMA,
        ],
    )
    def _combine(y_hbm, p0_hbm, p1_hbm, out_hbm,
                 p0_v, p1_v, bufa0, bufa1, bufb0, bufb1,
                 sa0, sa1, sb0, sb1, sw0, sw1):
        wid = lax.axis_index("s") * NC + lax.axis_index("c")
        pltpu.sync_copy(p0_hbm.at[wid], p0_v)
        pltpu.sync_copy(p1_hbm.at[wid], p1_v)
        bufa, bufb = (bufa0, bufa1), (bufb0, bufb1)
        sga, sgb, swb = (sa0, sa1), (sb0, sb1), (sw0, sw1)
        ga = [None, None]
        gb = [None, None]
        wcp = [None, None]
        ga[0] = pltpu.async_copy(y_hbm.at[p0_v.at[0]], bufa0, sa0)
        gb[0] = pltpu.async_copy(y_hbm.at[p1_v.at[0]], bufb0, sb0)
        for c in range(NCT):
            p, q = c & 1, (c + 1) & 1
            if c + 1 < NCT:
                if wcp[q] is not None:
                    wcp[q].wait()
                ga[q] = pltpu.async_copy(
                    y_hbm.at[p0_v.at[c + 1]], bufa[q], sga[q])
                gb[q] = pltpu.async_copy(
                    y_hbm.at[p1_v.at[c + 1]], bufb[q], sgb[q])
            ga[p].wait()
            gb[p].wait()
            ba, bb = bufa[p], bufb[p]

            def vec(j, carry, ba=ba, bb=bb):
                i = j >> 7
                col = pl.multiple_of((j & 127) << 4, 16)
                ba[i, pl.ds(col, 16)] = (
                    ba[i, pl.ds(col, 16)] + bb[i, pl.ds(col, 16)])
                return carry

            lax.fori_loop(0, CT * (DIM // 16), vec, 0, unroll=8)
            wcp[p] = pltpu.async_copy(
                ba, out_hbm.at[pl.ds(wid * TW + c * CT, CT)], swb[p])
        wcp[0].wait()
        wcp[1].wait()

    return _combine


def kernel(x, token_mask, weights, indices, gate_and_up_projs, down_projs):
    src_row, block_expert, pair_pos, pw = _plan(indices, token_mask, weights)
    a = _build_dispatch()(x, src_row.reshape(NW, NGC, GC))
    y = _grouped_gemm(block_expert, a, gate_and_up_projs, down_projs, pw)
    p0 = pair_pos[:, 0].reshape(NW, NCT, CT)
    p1 = pair_pos[:, 1].reshape(NW, NCT, CT)
    out = _build_combine()(y, p0, p1)
    return out
